# baseline (speedup 1.0000x reference)
import jax
import jax.numpy as jnp
from jax.experimental import pallas as pl
from jax.experimental.pallas import tpu as pltpu

B = 8
K = 128
N = 100000
CH = 5000
NCHUNK = N // CH
NBUF = 3

_TINY = 1.1754943508222875e-38


def _threefry2x32_bits(idx):
    rot0 = (13, 15, 26, 6)
    rot1 = (17, 29, 16, 24)
    ks0 = jnp.uint32(0)
    ks1 = jnp.uint32(1)
    ks2 = ks0 ^ ks1 ^ jnp.uint32(0x1BD11BDA)
    ks = (ks0, ks1, ks2)

    x0 = jnp.zeros_like(idx) + ks0
    x1 = idx + ks1

    def rotl(x, d):
        return (x << jnp.uint32(d)) | (x >> jnp.uint32(32 - d))

    for blk, rots in enumerate((rot0, rot1, rot0, rot1, rot0)):
        for r in rots:
            x0 = x0 + x1
            x1 = x0 ^ rotl(x1, r)
        x0 = x0 + ks[(blk + 1) % 3]
        x1 = x1 + ks[(blk + 2) % 3] + jnp.uint32(blk + 1)
    return x0 ^ x1


def _chunk_copy(wt_ref, buf_ref, sem, k, slot):
    return pltpu.make_async_copy(
        wt_ref.at[pl.ds(k * CH, CH), :],
        buf_ref.at[slot],
        sem.at[slot],
    )


def _sample_kernel(state_ref, wt_ref, b_ref, out_ref, buf_ref, m_sc, i_sc,
                   sem):
    for i in range(NBUF):
        _chunk_copy(wt_ref, buf_ref, sem, i, i).start()

    def body(k, _):
        slot = jax.lax.rem(k, NBUF)
        _chunk_copy(wt_ref, buf_ref, sem, k, slot).wait()
        w_chunk = buf_ref[slot]

        logits = jax.lax.dot_general(
            state_ref[...], w_chunk,
            dimension_numbers=(((1,), (1,)), ((), ())),
            preferred_element_type=jnp.float32)
        logits = logits + b_ref[pl.ds(k, 1)].reshape(1, CH)

        col = k * CH + jax.lax.broadcasted_iota(jnp.int32, (B, CH), 1)
        row = jax.lax.broadcasted_iota(jnp.int32, (B, CH), 0)
        lin = (row * N + col).astype(jnp.uint32)
        bits = _threefry2x32_bits(lin)
        fb = pltpu.bitcast((bits >> jnp.uint32(9)) | jnp.uint32(0x3F800000),
                           jnp.float32) - jnp.float32(1.0)
        tiny = jnp.float32(_TINY)
        u = jnp.maximum(tiny, fb + tiny)
        score = -jnp.log(-jnp.log(u)) + logits

        m = jnp.max(score, axis=1, keepdims=True)
        cand = jnp.where(score == m, col, jnp.int32(2**31 - 1))
        idx = jnp.min(cand, axis=1, keepdims=True)

        better = (m > m_sc[...]) | (k == 0)
        m_sc[...] = jnp.where(better, m, m_sc[...])
        i_sc[...] = jnp.where(better, idx, i_sc[...])

        @pl.when(k + NBUF < NCHUNK)
        def _next():
            _chunk_copy(wt_ref, buf_ref, sem, k + NBUF, slot).start()

        return 0

    jax.lax.fori_loop(0, NCHUNK, body, 0)
    out_ref[...] = i_sc[...]


@jax.jit
def kernel(state, W, b):
    wt = W.T
    b3 = b.reshape(NCHUNK, 1, CH)
    out = pl.pallas_call(
        _sample_kernel,
        in_specs=[
            pl.BlockSpec(memory_space=pltpu.VMEM),
            pl.BlockSpec(memory_space=pl.ANY),
            pl.BlockSpec(memory_space=pltpu.VMEM),
        ],
        out_specs=pl.BlockSpec(memory_space=pltpu.VMEM),
        out_shape=jax.ShapeDtypeStruct((B, 1), jnp.int32),
        scratch_shapes=[
            pltpu.VMEM((NBUF, CH, K), jnp.float32),
            pltpu.VMEM((B, 1), jnp.float32),
            pltpu.VMEM((B, 1), jnp.int32),
            pltpu.SemaphoreType.DMA((NBUF,)),
        ],
    )(state, wt, b3)
    return out.reshape(B)
